# baseline (speedup 1.0000x reference)
import functools

import jax
import jax.numpy as jnp
from jax import lax
from jax.experimental import pallas as pl
from jax.experimental.pallas import tpu as pltpu
from jax.experimental.pallas import tpu_sc as plsc

_D = 768
_B = 4
_S = 4096
_N = _B * _S
_NC = 2
_NS = 16
_NW = _NC * _NS
_SPAN = _S // _NW
_ROWS_PER_W = _SPAN * _B
_CHUNK = 16
_NCHUNKS = _ROWS_PER_W // _CHUNK
_CHUNKS_PER_B = _SPAN // _CHUNK
_LANES = 16
_VECS_PER_ROW = _D // _LANES


def _make_sc_kernel():
    mesh = plsc.VectorSubcoreMesh(core_axis_name="c", subcore_axis_name="s")

    @functools.partial(
        pl.kernel,
        out_type=jax.ShapeDtypeStruct((_N, _D), jnp.float32),
        mesh=mesh,
        scratch_types=[
            pltpu.VMEM((_ROWS_PER_W,), jnp.int32),
            pltpu.VMEM((_SPAN, _D), jnp.float32),
            pltpu.VMEM((_CHUNK, _D), jnp.float32),
            pltpu.VMEM((_CHUNK, _D), jnp.float32),
            pltpu.SemaphoreType.DMA,
            pltpu.SemaphoreType.DMA,
        ],
    )
    def body(ids_hbm, word_hbm, pos_hbm, out_hbm, idx_v, pos_v, buf0, buf1,
             sem0, sem1):
        wid = lax.axis_index("s") * _NC + lax.axis_index("c")
        s0 = wid * _SPAN
        for b in range(_B):
            pltpu.sync_copy(
                ids_hbm.at[pl.ds(b * _S + s0, _SPAN)],
                idx_v.at[pl.ds(b * _SPAN, _SPAN)],
            )
        pltpu.async_copy(word_hbm.at[idx_v.at[pl.ds(0, _CHUNK)]], buf0, sem0)
        pltpu.sync_copy(pos_hbm.at[pl.ds(s0, _SPAN)], pos_v)

        def do_chunk(c, buf):
            b_idx = c // _CHUNKS_PER_B
            j0 = (c % _CHUNKS_PER_B) * _CHUNK

            def row_step(r, carry):
                for k in range(_VECS_PER_ROW):
                    plsc.addupdate(
                        buf.at[r, pl.ds(k * _LANES, _LANES)],
                        pos_v[j0 + r, pl.ds(k * _LANES, _LANES)],
                    )
                return carry

            lax.fori_loop(0, _CHUNK, row_step, 0, unroll=False)
            out_off = b_idx * _S + s0 + j0
            pltpu.sync_copy(buf, out_hbm.at[pl.ds(out_off, _CHUNK)])

        def loop_body(i, carry):
            c0 = i * 2
            g1 = pltpu.async_copy(
                word_hbm.at[idx_v.at[pl.ds((c0 + 1) * _CHUNK, _CHUNK)]],
                buf1, sem1,
            )
            pltpu.make_async_copy(word_hbm.at[pl.ds(0, _CHUNK)], buf0, sem0).wait()
            do_chunk(c0, buf0)

            @pl.when(c0 + 2 < _NCHUNKS)
            def _():
                pltpu.async_copy(
                    word_hbm.at[idx_v.at[pl.ds((c0 + 2) * _CHUNK, _CHUNK)]],
                    buf0, sem0,
                )

            g1.wait()
            do_chunk(c0 + 1, buf1)
            return carry

        lax.fori_loop(0, _NCHUNKS // 2, loop_body, 0, unroll=False)

    return body


_sc_kernel = _make_sc_kernel()


@jax.jit
def kernel(input_ids, word_embeddings, position_embeddings):
    ids_flat = jnp.reshape(input_ids.astype(jnp.int32), (_N,))
    out = _sc_kernel(ids_flat, word_embeddings, position_embeddings)
    return jnp.reshape(out, (_B, _S, _D))


# speedup vs baseline: 1.6138x; 1.6138x over previous
import functools

import jax
import jax.numpy as jnp
from jax import lax
from jax.experimental import pallas as pl
from jax.experimental.pallas import tpu as pltpu
from jax.experimental.pallas import tpu_sc as plsc

_D = 768
_B = 4
_S = 4096
_N = _B * _S
_NC = 2
_NS = 16
_NW = _NC * _NS
_ROWS_PER_W = _N // _NW
_CHUNK = 32
_NCHUNKS = _ROWS_PER_W // _CHUNK
_LANES = 16
_VECS_PER_ROW = _D // _LANES


def _make_sc_kernel():
    mesh = plsc.VectorSubcoreMesh(core_axis_name="c", subcore_axis_name="s")

    @functools.partial(
        pl.kernel,
        out_type=jax.ShapeDtypeStruct((_N, _D), jnp.float32),
        mesh=mesh,
        scratch_types=[
            pltpu.VMEM((_ROWS_PER_W,), jnp.int32),
            pltpu.VMEM((_CHUNK, _D), jnp.float32),
            pltpu.VMEM((_CHUNK, _D), jnp.float32),
            pltpu.VMEM((_CHUNK, _D), jnp.float32),
            pltpu.VMEM((_CHUNK, _D), jnp.float32),
            pltpu.SemaphoreType.DMA,
            pltpu.SemaphoreType.DMA,
            pltpu.SemaphoreType.DMA,
            pltpu.SemaphoreType.DMA,
        ],
    )
    def body(ids_hbm, word_hbm, pos_hbm, out_hbm, idx_v,
             rows0, rows1, pos0, pos1, sg0, sg1, sp0, sp1):
        wid = lax.axis_index("s") * _NC + lax.axis_index("c")
        base = wid * _ROWS_PER_W
        pos_base = lax.rem(base, _S)
        pltpu.sync_copy(ids_hbm.at[pl.ds(base, _ROWS_PER_W)], idx_v)

        def start(c, rows, pos, sg, sp):
            off = c * _CHUNK
            pltpu.async_copy(
                word_hbm.at[idx_v.at[pl.ds(off, _CHUNK)]], rows, sg)
            pltpu.async_copy(
                pos_hbm.at[pl.ds(pos_base + off, _CHUNK)], pos, sp)

        def finish(c, rows, pos, sg, sp):
            pltpu.make_async_copy(word_hbm.at[pl.ds(0, _CHUNK)], rows, sg).wait()
            pltpu.make_async_copy(pos_hbm.at[pl.ds(0, _CHUNK)], pos, sp).wait()

            def row_step(r, carry):
                for k in range(_VECS_PER_ROW):
                    plsc.addupdate(
                        rows.at[r, pl.ds(k * _LANES, _LANES)],
                        pos[r, pl.ds(k * _LANES, _LANES)],
                    )
                return carry

            lax.fori_loop(0, _CHUNK, row_step, 0, unroll=4)
            pltpu.sync_copy(rows, out_hbm.at[pl.ds(base + c * _CHUNK, _CHUNK)])

        start(0, rows0, pos0, sg0, sp0)

        def loop_body(i, carry):
            c0 = i * 2
            start(c0 + 1, rows1, pos1, sg1, sp1)
            finish(c0, rows0, pos0, sg0, sp0)

            @pl.when(c0 + 2 < _NCHUNKS)
            def _():
                start(c0 + 2, rows0, pos0, sg0, sp0)

            finish(c0 + 1, rows1, pos1, sg1, sp1)
            return carry

        lax.fori_loop(0, _NCHUNKS // 2, loop_body, 0, unroll=False)

    return body


_sc_kernel = _make_sc_kernel()


@jax.jit
def kernel(input_ids, word_embeddings, position_embeddings):
    ids_flat = jnp.reshape(input_ids.astype(jnp.int32), (_N,))
    out = _sc_kernel(ids_flat, word_embeddings, position_embeddings)
    return jnp.reshape(out, (_B, _S, _D))
